# baseline (speedup 1.0000x reference)
import functools

import jax
import jax.numpy as jnp
from jax import lax
from jax.experimental import pallas as pl
from jax.experimental.pallas import tpu as pltpu
from jax.experimental.pallas import tpu_sc as plsc

N_NODES = 10000
N_EDGES = 320000
N_CITIES = 100
N_ETH = 128

NC = 2
NS = 16
L = 16
NW = NC * NS

K = 128
EPW = -(-N_EDGES // NW)
CHUNKS = -(-EPW // K)
CP = CHUNKS + 1
ROWS_PAD = 10112
ROW_CHUNKS = ROWS_PAD // K
P2_T = -(-ROW_CHUNKS // NS)
CB = 8
GRP = K // L


def _sc_body(pop, coef, n2c, src3, dst3, lap3,
             out,
             acc, rows0, rows1, lapb0, didx0, sidx0,
             sG0, sS0):
    c = lax.axis_index("c")
    s = lax.axis_index("s")
    w = c * NS + s

    zvec = jnp.zeros((L,), jnp.float32)

    def zero_row(r, _):
        for b in range(CB):
            rows0[r, pl.ds(b * L, L)] = zvec
        return 0

    lax.fori_loop(0, K, zero_row, 0)
    for t in range(P2_T):
        chunk = s + NS * t

        @pl.when(chunk < ROW_CHUNKS)
        def _():
            pltpu.sync_copy(rows0, acc.at[pl.ds(chunk * K, K)])
    plsc.subcore_barrier()

    def chunk_body(i, _):
        pltpu.sync_copy(dst3.at[w, i], didx0)
        pltpu.sync_copy(lap3.at[w, i], lapb0)
        pltpu.sync_copy(src3.at[w, i], sidx0)
        pltpu.sync_copy(pop.at[didx0], rows0)

        def scale_grp(g, _):
            lv16 = lapb0[pl.ds(g * L, L)]
            for u in range(L):
                e = g * L + u
                lv = lv16[u]
                for b in range(CB):
                    sl = (e, pl.ds(b * L, L))
                    rows0[sl] = rows0[sl] * lv
            return 0

        lax.fori_loop(0, GRP, scale_grp, 0)
        pltpu.sync_copy(rows0, acc.at[sidx0], add=True)
        return 0

    lax.fori_loop(0, CHUNKS, chunk_body, 0)
    plsc.subcore_barrier()

    def mul_body(r4, _):
        for u in range(4):
            r = r4 * 4 + u
            for b in range(CB):
                sl = (r, pl.ds(b * L, L))
                rows0[sl] = rows0[sl] * rows1[sl]
        return 0

    def p2_chunk(t, _):
        chunk = s + NS * t

        @pl.when(chunk < ROW_CHUNKS)
        def _():
            r0 = chunk * K
            pltpu.sync_copy(n2c.at[pl.ds(r0, K)], didx0)
            pltpu.async_copy(coef.at[didx0], rows1, sG0)
            pltpu.sync_copy(acc.at[pl.ds(r0, K)], rows0)
            pltpu.make_async_copy(coef.at[didx0], rows1, sG0).wait()
            lax.fori_loop(0, K // 4, mul_body, 0)
            pltpu.sync_copy(rows0, out.at[pl.ds(c * ROWS_PAD + r0, K)])
        return 0

    lax.fori_loop(0, P2_T, p2_chunk, 0)


def _add_body(a_ref, b_ref, o_ref):
    o_ref[...] = a_ref[...] + b_ref[...]


@jax.jit
def _run(pop, coef, n2c, src3, dst3, lap3):
    f32 = jnp.float32
    i32 = jnp.int32
    kern = pl.kernel(
        _sc_body,
        out_type=jax.ShapeDtypeStruct((NC * ROWS_PAD, N_ETH), f32),
        mesh=plsc.VectorSubcoreMesh(
            core_axis_name="c", subcore_axis_name="s",
            num_cores=NC, num_subcores=NS,
        ),
        scratch_types=[
            pltpu.VMEM_SHARED((ROWS_PAD, N_ETH), f32),
            pltpu.VMEM((K, N_ETH), f32),
            pltpu.VMEM((K, N_ETH), f32),
            pltpu.VMEM((K,), f32),
            pltpu.VMEM((K,), i32),
            pltpu.VMEM((K,), i32),
        ] + [pltpu.SemaphoreType.DMA] * 2,
    )
    partial = kern(pop, coef, n2c, src3, dst3, lap3)

    final = pl.pallas_call(
        _add_body,
        out_shape=jax.ShapeDtypeStruct((ROWS_PAD, N_ETH), f32),
        grid=(ROWS_PAD // K,),
        in_specs=[
            pl.BlockSpec((K, N_ETH), lambda i: (i, 0)),
            pl.BlockSpec((K, N_ETH), lambda i: (i + ROWS_PAD // K, 0)),
        ],
        out_specs=pl.BlockSpec((K, N_ETH), lambda i: (i, 0)),
    )(partial, partial)
    return final


def kernel(population, diffusion_coef, lap_values, src, dst, node_to_city):
    n2c = jnp.pad(node_to_city, (0, ROWS_PAD - N_NODES))
    pad_e = NW * EPW - N_EDGES
    src_p = jnp.pad(src, (0, pad_e), constant_values=N_NODES)
    dst_p = jnp.pad(dst, (0, pad_e))
    lap_p = jnp.pad(lap_values, (0, pad_e))
    src3 = jnp.pad(src_p.reshape(NW, EPW), ((0, 0), (0, CP * K - EPW)),
                   constant_values=N_NODES).reshape(NW, CP, K)
    dst3 = jnp.pad(dst_p.reshape(NW, EPW),
                   ((0, 0), (0, CP * K - EPW))).reshape(NW, CP, K)
    lap3 = jnp.pad(lap_p.reshape(NW, EPW),
                   ((0, 0), (0, CP * K - EPW))).reshape(NW, CP, K)
    final = _run(population, diffusion_coef, n2c, src3, dst3, lap3)
    return final[:N_NODES]


# speedup vs baseline: 1.1372x; 1.1372x over previous
import functools

import jax
import jax.numpy as jnp
from jax import lax
from jax.experimental import pallas as pl
from jax.experimental.pallas import tpu as pltpu
from jax.experimental.pallas import tpu_sc as plsc

N_NODES = 10000
N_EDGES = 320000
N_CITIES = 100
N_ETH = 128

NC = 2
NS = 16
L = 16
NW = NC * NS

K = 128
EPW = -(-N_EDGES // NW)
CHUNKS = -(-EPW // K)
CP = CHUNKS + 1
ROWS_PAD = 10112
ROW_CHUNKS = ROWS_PAD // K
P2_T = -(-ROW_CHUNKS // NS)
CB = 8
GRP = K // L


def _sc_body(pop, coef, n2c, src3, dst3, lap3,
             out,
             acc, rows0, rows1, lapb0, didx0, sidx0,
             sG0, sS0):
    c = lax.axis_index("c")
    s = lax.axis_index("s")
    w = c * NS + s

    zvec = jnp.zeros((L,), jnp.float32)

    def zero_row(r, _):
        for b in range(CB):
            rows0[r, pl.ds(b * L, L)] = zvec
        return 0

    lax.fori_loop(0, K, zero_row, 0)
    for t in range(P2_T):
        chunk = s + NS * t

        @pl.when(chunk < ROW_CHUNKS)
        def _():
            pltpu.sync_copy(rows0, acc.at[pl.ds(chunk * K, K)])
    plsc.subcore_barrier()

    def chunk_body(i, _):
        pltpu.sync_copy(dst3.at[w, i], didx0)
        pltpu.async_copy(pop.at[didx0], rows0, sG0)
        pltpu.sync_copy(lap3.at[w, i], lapb0)
        pltpu.sync_copy(src3.at[w, i], sidx0)
        pltpu.make_async_copy(pop.at[didx0], rows0, sG0).wait()

        def scale_grp(g, _):
            lv16 = lapb0[pl.ds(g * L, L)]
            for u in range(L):
                e = g * L + u
                lv = lv16[u]
                for b in range(CB):
                    sl = (e, pl.ds(b * L, L))
                    rows0[sl] = rows0[sl] * lv
            return 0

        lax.fori_loop(0, GRP, scale_grp, 0)
        pltpu.sync_copy(rows0, acc.at[sidx0], add=True)
        return 0

    lax.fori_loop(0, CHUNKS, chunk_body, 0)
    plsc.subcore_barrier()

    def mul_body(r4, _):
        for u in range(4):
            r = r4 * 4 + u
            for b in range(CB):
                sl = (r, pl.ds(b * L, L))
                rows0[sl] = rows0[sl] * rows1[sl]
        return 0

    def p2_chunk(t, _):
        chunk = s + NS * t

        @pl.when(chunk < ROW_CHUNKS)
        def _():
            r0 = chunk * K
            pltpu.sync_copy(n2c.at[pl.ds(r0, K)], didx0)
            pltpu.async_copy(coef.at[didx0], rows1, sG0)
            pltpu.sync_copy(acc.at[pl.ds(r0, K)], rows0)
            pltpu.make_async_copy(coef.at[didx0], rows1, sG0).wait()
            lax.fori_loop(0, K // 4, mul_body, 0)
            pltpu.sync_copy(rows0, out.at[pl.ds(c * ROWS_PAD + r0, K)])
        return 0

    lax.fori_loop(0, P2_T, p2_chunk, 0)


def _add_body(a_ref, b_ref, o_ref):
    o_ref[...] = a_ref[...] + b_ref[...]


@jax.jit
def _run(pop, coef, n2c, src3, dst3, lap3):
    f32 = jnp.float32
    i32 = jnp.int32
    kern = pl.kernel(
        _sc_body,
        out_type=jax.ShapeDtypeStruct((NC * ROWS_PAD, N_ETH), f32),
        mesh=plsc.VectorSubcoreMesh(
            core_axis_name="c", subcore_axis_name="s",
            num_cores=NC, num_subcores=NS,
        ),
        scratch_types=[
            pltpu.VMEM_SHARED((ROWS_PAD, N_ETH), f32),
            pltpu.VMEM((K, N_ETH), f32),
            pltpu.VMEM((K, N_ETH), f32),
            pltpu.VMEM((K,), f32),
            pltpu.VMEM((K,), i32),
            pltpu.VMEM((K,), i32),
        ] + [pltpu.SemaphoreType.DMA] * 2,
    )
    partial = kern(pop, coef, n2c, src3, dst3, lap3)

    final = pl.pallas_call(
        _add_body,
        out_shape=jax.ShapeDtypeStruct((ROWS_PAD, N_ETH), f32),
        grid=(ROWS_PAD // K,),
        in_specs=[
            pl.BlockSpec((K, N_ETH), lambda i: (i, 0)),
            pl.BlockSpec((K, N_ETH), lambda i: (i + ROWS_PAD // K, 0)),
        ],
        out_specs=pl.BlockSpec((K, N_ETH), lambda i: (i, 0)),
    )(partial, partial)
    return final


def kernel(population, diffusion_coef, lap_values, src, dst, node_to_city):
    n2c = jnp.pad(node_to_city, (0, ROWS_PAD - N_NODES))
    pad_e = NW * EPW - N_EDGES
    src_p = jnp.pad(src, (0, pad_e), constant_values=N_NODES)
    dst_p = jnp.pad(dst, (0, pad_e))
    lap_p = jnp.pad(lap_values, (0, pad_e))
    src3 = jnp.pad(src_p.reshape(NW, EPW), ((0, 0), (0, CP * K - EPW)),
                   constant_values=N_NODES).reshape(NW, CP, K)
    dst3 = jnp.pad(dst_p.reshape(NW, EPW),
                   ((0, 0), (0, CP * K - EPW))).reshape(NW, CP, K)
    lap3 = jnp.pad(lap_p.reshape(NW, EPW),
                   ((0, 0), (0, CP * K - EPW))).reshape(NW, CP, K)
    final = _run(population, diffusion_coef, n2c, src3, dst3, lap3)
    return final[:N_NODES]


# speedup vs baseline: 1.2213x; 1.0739x over previous
import functools

import jax
import jax.numpy as jnp
from jax import lax
from jax.experimental import pallas as pl
from jax.experimental.pallas import tpu as pltpu
from jax.experimental.pallas import tpu_sc as plsc

N_NODES = 10000
N_EDGES = 320000
N_CITIES = 100
N_ETH = 128

NC = 2
NS = 16
L = 16
NW = NC * NS

K = 128
EPW = -(-N_EDGES // NW)
CHUNKS = -(-EPW // K)
CP = CHUNKS + 1
ROWS_PAD = 10112
ROW_CHUNKS = ROWS_PAD // K
P2_T = -(-ROW_CHUNKS // NS)
CB = 8
GRP = K // L


def _sc_body(pop, coef, n2c, src3, dst3, lap3,
             out,
             acc, rows0, rows1, lapb0, didx0, sidx0,
             sG0, sS0):
    c = lax.axis_index("c")
    s = lax.axis_index("s")
    w = c * NS + s

    zvec = jnp.zeros((L,), jnp.float32)

    def zero_row(r, _):
        for b in range(CB):
            rows0[r, pl.ds(b * L, L)] = zvec
        return 0

    lax.fori_loop(0, K, zero_row, 0)
    for t in range(P2_T):
        chunk = s + NS * t

        @pl.when(chunk < ROW_CHUNKS)
        def _():
            pltpu.sync_copy(rows0, acc.at[pl.ds(chunk * K, K)])
    plsc.subcore_barrier()

    pltpu.sync_copy(src3.at[w, 0], sidx0)
    pltpu.make_async_copy(rows0, acc.at[sidx0], sS0).start(add=True)

    def chunk_body(i, _):
        pltpu.sync_copy(dst3.at[w, i], didx0)
        pltpu.make_async_copy(rows0, acc.at[sidx0], sS0).wait()
        pltpu.async_copy(pop.at[didx0], rows0, sG0)
        pltpu.sync_copy(lap3.at[w, i], lapb0)
        pltpu.sync_copy(src3.at[w, i], sidx0)
        pltpu.make_async_copy(pop.at[didx0], rows0, sG0).wait()

        def scale_grp(g, _):
            lv16 = lapb0[pl.ds(g * L, L)]
            for u in range(L):
                e = g * L + u
                lv = lv16[u]
                for b in range(CB):
                    sl = (e, pl.ds(b * L, L))
                    rows0[sl] = rows0[sl] * lv
            return 0

        lax.fori_loop(0, GRP, scale_grp, 0)
        pltpu.make_async_copy(rows0, acc.at[sidx0], sS0).start(add=True)
        return 0

    lax.fori_loop(0, CHUNKS, chunk_body, 0)
    pltpu.make_async_copy(rows0, acc.at[sidx0], sS0).wait()
    plsc.subcore_barrier()

    def mul_body(r4, _):
        for u in range(4):
            r = r4 * 4 + u
            for b in range(CB):
                sl = (r, pl.ds(b * L, L))
                rows0[sl] = rows0[sl] * rows1[sl]
        return 0

    def p2_chunk(t, _):
        chunk = s + NS * t

        @pl.when(chunk < ROW_CHUNKS)
        def _():
            r0 = chunk * K
            pltpu.sync_copy(n2c.at[pl.ds(r0, K)], didx0)
            pltpu.async_copy(coef.at[didx0], rows1, sG0)
            pltpu.sync_copy(acc.at[pl.ds(r0, K)], rows0)
            pltpu.make_async_copy(coef.at[didx0], rows1, sG0).wait()
            lax.fori_loop(0, K // 4, mul_body, 0)
            pltpu.sync_copy(rows0, out.at[pl.ds(c * ROWS_PAD + r0, K)])
        return 0

    lax.fori_loop(0, P2_T, p2_chunk, 0)


def _add_body(a_ref, b_ref, o_ref):
    o_ref[...] = a_ref[...] + b_ref[...]


@jax.jit
def _run(pop, coef, n2c, src3, dst3, lap3):
    f32 = jnp.float32
    i32 = jnp.int32
    kern = pl.kernel(
        _sc_body,
        out_type=jax.ShapeDtypeStruct((NC * ROWS_PAD, N_ETH), f32),
        mesh=plsc.VectorSubcoreMesh(
            core_axis_name="c", subcore_axis_name="s",
            num_cores=NC, num_subcores=NS,
        ),
        scratch_types=[
            pltpu.VMEM_SHARED((ROWS_PAD, N_ETH), f32),
            pltpu.VMEM((K, N_ETH), f32),
            pltpu.VMEM((K, N_ETH), f32),
            pltpu.VMEM((K,), f32),
            pltpu.VMEM((K,), i32),
            pltpu.VMEM((K,), i32),
        ] + [pltpu.SemaphoreType.DMA] * 2,
    )
    partial = kern(pop, coef, n2c, src3, dst3, lap3)

    final = pl.pallas_call(
        _add_body,
        out_shape=jax.ShapeDtypeStruct((ROWS_PAD, N_ETH), f32),
        grid=(ROWS_PAD // K,),
        in_specs=[
            pl.BlockSpec((K, N_ETH), lambda i: (i, 0)),
            pl.BlockSpec((K, N_ETH), lambda i: (i + ROWS_PAD // K, 0)),
        ],
        out_specs=pl.BlockSpec((K, N_ETH), lambda i: (i, 0)),
    )(partial, partial)
    return final


def kernel(population, diffusion_coef, lap_values, src, dst, node_to_city):
    n2c = jnp.pad(node_to_city, (0, ROWS_PAD - N_NODES))
    pad_e = NW * EPW - N_EDGES
    src_p = jnp.pad(src, (0, pad_e), constant_values=N_NODES)
    dst_p = jnp.pad(dst, (0, pad_e))
    lap_p = jnp.pad(lap_values, (0, pad_e))
    src3 = jnp.pad(src_p.reshape(NW, EPW), ((0, 0), (0, CP * K - EPW)),
                   constant_values=N_NODES).reshape(NW, CP, K)
    dst3 = jnp.pad(dst_p.reshape(NW, EPW),
                   ((0, 0), (0, CP * K - EPW))).reshape(NW, CP, K)
    lap3 = jnp.pad(lap_p.reshape(NW, EPW),
                   ((0, 0), (0, CP * K - EPW))).reshape(NW, CP, K)
    final = _run(population, diffusion_coef, n2c, src3, dst3, lap3)
    return final[:N_NODES]
